# baseline (speedup 1.0000x reference)
import functools

import jax
import jax.numpy as jnp
from jax import lax
from jax.experimental import pallas as pl
from jax.experimental.pallas import tpu as pltpu
from jax.experimental.pallas import tpu_sc as plsc

EMB_DIM = 64
TWO_DIM = 2 * EMB_DIM
BATCH = 16384
NREL = 1000
L = 16

NUM_CORES = 2
NUM_SUBCORES = 16
NW = NUM_CORES * NUM_SUBCORES
BPW = BATCH // NW
CHUNK = 128
NCHUNK = BPW // CHUNK

ROWS_HI = 64
ROWS_LO = 56
SPLIT_TILE = 13

SIN_C = (0.99999970703, -0.16666577215, 0.0083325581177,
         -0.0001981257552, 2.7040512122e-06, -2.0534244492e-08)
COS_C = (0.99999944371, -0.49999558229, 0.041661033519,
         -0.0013862749961, 2.425322989e-05, -2.219412982e-07)


def _rsqrt(x):
    i = lax.bitcast_convert_type(x, jnp.int32)
    i = jnp.int32(0x5F3759DF) - (i >> 1)
    y = lax.bitcast_convert_type(i, jnp.float32)
    xh = x * jnp.float32(-0.5)
    y = y * (jnp.float32(1.5) + xh * y * y)
    y = y * (jnp.float32(1.5) + xh * y * y)
    return y


def _sincos(x):
    z = x * x
    s = jnp.float32(SIN_C[5])
    for c in SIN_C[4::-1]:
        s = s * z + jnp.float32(c)
    s = s * x
    c_ = jnp.float32(COS_C[5])
    for c in COS_C[4::-1]:
        c_ = c_ * z + jnp.float32(c)
    return s, c_


def _sc_body(hid_hbm, tid_hbm, rid_hbm, ent_hbm, rel_hbm, out_hbm, trig_hbm,
             hid_v, tid_v, rid_v,
             head_a, tail_a, trig_a, head_b, tail_b, trig_b,
             phase_v, tloc_v, accs_v, out_v, sem):
    cid = lax.axis_index("c")
    sid = lax.axis_index("s")
    wid = sid * NUM_CORES + cid
    base = wid * BPW

    def fill_rows(nrows):
        def trig_row(r, carry):
            for j in range(EMB_DIM // L):
                x = phase_v[r, pl.ds(j * L, L)]
                s, c = _sincos(x)
                tloc_v[r, pl.ds(j * L, L)] = c
                tloc_v[r, pl.ds(EMB_DIM + j * L, L)] = s
            return carry
        lax.fori_loop(0, nrows, trig_row, 0)

    @pl.when(sid < SPLIT_TILE)
    def _():
        roff = sid * ROWS_HI
        pltpu.sync_copy(rel_hbm.at[pl.ds(roff, ROWS_HI)], phase_v)
        fill_rows(ROWS_HI)
        pltpu.sync_copy(tloc_v, trig_hbm.at[pl.ds(roff, ROWS_HI)])

    @pl.when(sid >= SPLIT_TILE)
    def _():
        roff = sid * ROWS_LO + SPLIT_TILE * (ROWS_HI - ROWS_LO)
        pltpu.sync_copy(rel_hbm.at[pl.ds(roff, ROWS_LO)],
                        phase_v.at[pl.ds(0, ROWS_LO)])
        fill_rows(ROWS_LO)
        pltpu.sync_copy(tloc_v.at[pl.ds(0, ROWS_LO)],
                        trig_hbm.at[pl.ds(roff, ROWS_LO)])

    plsc.subcore_barrier()

    pltpu.sync_copy(hid_hbm.at[pl.ds(base, BPW)], hid_v)
    pltpu.sync_copy(tid_hbm.at[pl.ds(base, BPW)], tid_v)
    pltpu.sync_copy(rid_hbm.at[pl.ds(base, BPW)], rid_v)

    lane = lax.iota(jnp.int32, L)
    idx0 = lane * L

    def issue(cbase, hb, tb, gb):
        c1 = pltpu.async_copy(ent_hbm.at[hid_v.at[pl.ds(cbase, CHUNK)]],
                              hb, sem)
        c2 = pltpu.async_copy(ent_hbm.at[tid_v.at[pl.ds(cbase, CHUNK)]],
                              tb, sem)
        c3 = pltpu.async_copy(trig_hbm.at[rid_v.at[pl.ds(cbase, CHUNK)]],
                              gb, sem)
        return (c1, c2, c3)

    bufs = ((head_a, tail_a, trig_a), (head_b, tail_b, trig_b))
    pend = issue(0, *bufs[0])
    for ch in range(NCHUNK):
        cbase = ch * CHUNK
        head_v, tail_v, trig_v = bufs[ch % 2]
        cur = pend
        if ch + 1 < NCHUNK:
            pend = issue(cbase + CHUNK, *bufs[(ch + 1) % 2])
        for c in cur:
            c.wait()

        def group_body(g, carry2, head_v=head_v, tail_v=tail_v,
                       trig_v=trig_v, cbase=cbase):
            for k in range(L):
                r = g * L + k
                acc = jnp.zeros((L,), jnp.float32)
                for j in range(EMB_DIM // L):
                    off = j * L
                    hre = head_v[r, pl.ds(off, L)]
                    him = head_v[r, pl.ds(EMB_DIM + off, L)]
                    tre = tail_v[r, pl.ds(off, L)]
                    tim = tail_v[r, pl.ds(EMB_DIM + off, L)]
                    cosv = trig_v[r, pl.ds(off, L)]
                    sinv = trig_v[r, pl.ds(EMB_DIM + off, L)]
                    dre = hre * cosv - him * sinv - tre
                    dim_ = hre * sinv + him * cosv - tim
                    x = dre * dre + dim_ * dim_ + jnp.float32(1e-8)
                    acc = acc + x * _rsqrt(x)
                accs_v[pl.ds(k * L, L)] = acc
            tot = plsc.load_gather(accs_v, [idx0])
            for d in range(1, L):
                tot = tot + plsc.load_gather(accs_v, [idx0 + d])
            out_v[pl.ds(cbase + g * L, L)] = -tot
            return carry2

        lax.fori_loop(0, CHUNK // L, group_body, 0)

    pltpu.sync_copy(out_v, out_hbm.at[pl.ds(base, BPW)])


_sc_kernel = functools.partial(
    pl.kernel,
    mesh=plsc.VectorSubcoreMesh(core_axis_name="c", subcore_axis_name="s"),
    out_type=(jax.ShapeDtypeStruct((BATCH,), jnp.float32),
              jax.ShapeDtypeStruct((NREL, TWO_DIM), jnp.float32)),
    compiler_params=pltpu.CompilerParams(needs_layout_passes=False),
    scratch_types=[
        pltpu.VMEM((BPW,), jnp.int32),
        pltpu.VMEM((BPW,), jnp.int32),
        pltpu.VMEM((BPW,), jnp.int32),
        pltpu.VMEM((CHUNK, TWO_DIM), jnp.float32),
        pltpu.VMEM((CHUNK, TWO_DIM), jnp.float32),
        pltpu.VMEM((CHUNK, TWO_DIM), jnp.float32),
        pltpu.VMEM((CHUNK, TWO_DIM), jnp.float32),
        pltpu.VMEM((CHUNK, TWO_DIM), jnp.float32),
        pltpu.VMEM((CHUNK, TWO_DIM), jnp.float32),
        pltpu.VMEM((ROWS_HI, EMB_DIM), jnp.float32),
        pltpu.VMEM((ROWS_HI, TWO_DIM), jnp.float32),
        pltpu.VMEM((L * L,), jnp.float32),
        pltpu.VMEM((BPW,), jnp.float32),
        pltpu.SemaphoreType.DMA,
    ],
)(_sc_body)


def kernel(head_ids, relation_ids, tail_ids, entity_emb, relation_emb):
    scores, _ = _sc_kernel(
        head_ids.astype(jnp.int32),
        tail_ids.astype(jnp.int32),
        relation_ids.astype(jnp.int32),
        entity_emb,
        relation_emb,
    )
    return scores
